# baseline (speedup 1.0000x reference)
import functools

import jax
import jax.numpy as jnp
from jax import lax
from jax.experimental import pallas as pl
from jax.experimental.pallas import tpu as pltpu
from jax.experimental.pallas import tpu_sc as plsc


@functools.lru_cache(maxsize=None)
def _build_sc_kernel(q_len, k_len, in_dim, out_dim):
    info = plsc.get_sparse_core_info()
    num_cores, num_subcores, lanes = (
        info.num_cores, info.num_subcores, info.num_lanes)
    num_workers = num_cores * num_subcores
    rows_per_worker = q_len // num_workers
    shifts = 8
    max_index = (in_dim - 1) // 2
    cols = q_len + k_len - 1
    per_tile = ((cols + shifts + num_subcores * lanes - 1)
                // (num_subcores * lanes)) * lanes
    cols_pad = per_tile * num_subcores

    mesh = plsc.VectorSubcoreMesh(core_axis_name="c", subcore_axis_name="s")

    @functools.partial(
        pl.kernel,
        mesh=mesh,
        compiler_params=pltpu.CompilerParams(
            use_tc_tiling_on_sc=False, needs_layout_passes=False),
        out_type=(
            jax.ShapeDtypeStruct((q_len, out_dim, k_len), jnp.float32),
            jax.ShapeDtypeStruct(
                (num_cores, shifts, out_dim, cols_pad), jnp.float32),
        ),
        scratch_types=[
            pltpu.VMEM((out_dim * in_dim,), jnp.float32),
            pltpu.VMEM((out_dim, per_tile), jnp.float32),
            pltpu.SemaphoreType.DMA,
        ],
    )
    def rel_pos_kernel(emb_t_hbm, out_hbm, yt_hbm, et_v, build_v, sem):
        cid = lax.axis_index("c")
        sid = lax.axis_index("s")
        wid = sid * num_cores + cid
        region = wid // shifts
        phase = wid - region * shifts
        ibase = region * (q_len // (num_workers // shifts)) + phase

        pltpu.sync_copy(emb_t_hbm, et_v)

        m0 = pl.multiple_of(sid * per_tile, lanes)
        for d in range(shifts):

            def build_chunk(t, carry, d=d):
                m = m0 + t * lanes + lax.iota(jnp.int32, lanes)
                v = (q_len - 1) - m - d
                col = jnp.clip(v, -max_index, max_index) + max_index
                for e in range(out_dim):
                    vals = plsc.load_gather(et_v, [col + e * in_dim])
                    build_v[e, pl.ds(t * lanes, lanes)] = vals
                return carry

            lax.fori_loop(0, per_tile // lanes, build_chunk, 0)
            pltpu.sync_copy(build_v, yt_hbm.at[cid, d, :, pl.ds(m0, per_tile)])
        plsc.subcore_barrier()

        start_hi = q_len - 1 - ibase
        d = start_hi % shifts
        lag = 8
        handles = []
        for t in range(rows_per_worker):
            start = pl.multiple_of(start_hi - shifts * t - d, shifts)
            h = pltpu.async_copy(
                yt_hbm.at[cid, d, :, pl.ds(start, k_len)],
                out_hbm.at[ibase + shifts * t],
                sem,
            )
            handles.append(h)
            if t >= lag:
                handles[t - lag].wait()
        for h in handles[rows_per_worker - lag:]:
            h.wait()

    return rel_pos_kernel


def kernel(q, k, embeddings):
    q_len = q.shape[1]
    k_len = k.shape[1]
    in_dim, out_dim = embeddings.shape
    out_t, _ = _build_sc_kernel(q_len, k_len, in_dim, out_dim)(
        embeddings.T.reshape(-1))
    return jnp.swapaxes(out_t, 1, 2)


# speedup vs baseline: 26.5367x; 26.5367x over previous
import functools

import jax
import jax.numpy as jnp
from jax import lax
from jax.experimental import pallas as pl
from jax.experimental.pallas import tpu as pltpu
from jax.experimental.pallas import tpu_sc as plsc


@functools.lru_cache(maxsize=None)
def _build_sc_kernel(q_len, k_len, in_dim, out_dim):
    info = plsc.get_sparse_core_info()
    num_cores, num_subcores, lanes = (
        info.num_cores, info.num_subcores, info.num_lanes)
    num_workers = num_cores * num_subcores
    rows_per_worker = q_len // num_workers
    phases = 8
    regions = num_workers // phases
    region_rows = q_len // regions
    stride_span = phases * (rows_per_worker - 1)
    window = k_len + stride_span
    window_pad = ((window + lanes - 1) // lanes) * lanes
    max_index = (in_dim - 1) // 2

    mesh = plsc.VectorSubcoreMesh(core_axis_name="c", subcore_axis_name="s")

    @functools.partial(
        pl.kernel,
        mesh=mesh,
        compiler_params=pltpu.CompilerParams(
            use_tc_tiling_on_sc=False, needs_layout_passes=False),
        out_type=jax.ShapeDtypeStruct((q_len, out_dim, k_len), jnp.float32),
        scratch_types=[
            pltpu.VMEM((out_dim * in_dim,), jnp.float32),
            pltpu.VMEM((out_dim, window_pad), jnp.float32),
            pltpu.SemaphoreType.DMA,
            pltpu.SemaphoreType.DMA,
        ],
    )
    def rel_pos_kernel(emb_t_hbm, out_hbm, et_v, ywt_v, sem0, sem1):
        wid = lax.axis_index("s") * num_cores + lax.axis_index("c")
        region = wid // phases
        phase = wid - region * phases
        ibase = region * region_rows + phase

        pltpu.sync_copy(emb_t_hbm, et_v)

        def build_chunk(t, carry):
            m = t * lanes + lax.iota(jnp.int32, lanes)
            v = stride_span + ibase - m
            col = jnp.clip(v, -max_index, max_index) + max_index
            for e in range(out_dim):
                vals = plsc.load_gather(et_v, [col + e * in_dim])
                ywt_v[e, pl.ds(t * lanes, lanes)] = vals
            return carry

        lax.fori_loop(0, window_pad // lanes, build_chunk, 0)

        lag = 8
        half = out_dim // 2
        sems = (sem0, sem1)
        handles = ([], [])
        for t in range(rows_per_worker):
            off = stride_span - phases * t
            for hf in range(2):
                h = pltpu.async_copy(
                    ywt_v.at[pl.ds(hf * half, half), pl.ds(off, k_len)],
                    out_hbm.at[ibase + phases * t, pl.ds(hf * half, half)],
                    sems[hf],
                )
                handles[hf].append(h)
                if t >= lag:
                    handles[hf][t - lag].wait()
        for hs in handles:
            for h in hs[rows_per_worker - lag:]:
                h.wait()

    return rel_pos_kernel


def kernel(q, k, embeddings):
    q_len = q.shape[1]
    k_len = k.shape[1]
    in_dim, out_dim = embeddings.shape
    out_t = _build_sc_kernel(q_len, k_len, in_dim, out_dim)(
        embeddings.T.reshape(-1))
    return jnp.swapaxes(out_t, 1, 2)


# speedup vs baseline: 27.3096x; 1.0291x over previous
import functools

import jax
import jax.numpy as jnp
from jax import lax
from jax.experimental import pallas as pl
from jax.experimental.pallas import tpu as pltpu
from jax.experimental.pallas import tpu_sc as plsc


@functools.lru_cache(maxsize=None)
def _build_sc_kernel(q_len, k_len, in_dim, out_dim):
    info = plsc.get_sparse_core_info()
    num_cores, num_subcores, lanes = (
        info.num_cores, info.num_subcores, info.num_lanes)
    num_workers = num_cores * num_subcores
    rows_per_worker = q_len // num_workers
    phases = 8
    regions = num_workers // phases
    region_rows = q_len // regions
    stride_span = phases * (rows_per_worker - 1)
    window = k_len + stride_span
    window_pad = ((window + lanes - 1) // lanes) * lanes
    max_index = (in_dim - 1) // 2

    mesh = plsc.VectorSubcoreMesh(core_axis_name="c", subcore_axis_name="s")

    @functools.partial(
        pl.kernel,
        mesh=mesh,
        compiler_params=pltpu.CompilerParams(
            use_tc_tiling_on_sc=False, needs_layout_passes=False),
        out_type=jax.ShapeDtypeStruct((q_len, out_dim, k_len), jnp.float32),
        scratch_types=[
            pltpu.VMEM((out_dim * in_dim,), jnp.float32),
            pltpu.VMEM((out_dim, window_pad), jnp.float32),
            pltpu.SemaphoreType.DMA,
        ],
    )
    def rel_pos_kernel(emb_t_hbm, out_hbm, et_v, ywt_v, sem):
        wid = lax.axis_index("s") * num_cores + lax.axis_index("c")
        region = wid // phases
        phase = wid - region * phases
        ibase = region * region_rows + phase

        pltpu.sync_copy(emb_t_hbm, et_v)

        @plsc.parallel_loop(0, window_pad // lanes, 1, unroll=4)
        def build_chunk(t):
            m = t * lanes + lax.iota(jnp.int32, lanes)
            v = stride_span + ibase - m
            col = jnp.clip(v, -max_index, max_index) + max_index
            for e in range(out_dim):
                vals = plsc.load_gather(et_v, [col + e * in_dim])
                ywt_v[e, pl.ds(t * lanes, lanes)] = vals

        lag = 8
        handles = []
        for t in range(rows_per_worker):
            h = pltpu.async_copy(
                ywt_v.at[:, pl.ds(stride_span - phases * t, k_len)],
                out_hbm.at[ibase + phases * t],
                sem,
            )
            handles.append(h)
            if t >= lag:
                handles[t - lag].wait()
        for h in handles[rows_per_worker - lag:]:
            h.wait()

    return rel_pos_kernel


def kernel(q, k, embeddings):
    q_len = q.shape[1]
    k_len = k.shape[1]
    in_dim, out_dim = embeddings.shape
    out_t = _build_sc_kernel(q_len, k_len, in_dim, out_dim)(
        embeddings.T.reshape(-1))
    return jnp.swapaxes(out_t, 1, 2)
